# Initial kernel scaffold; baseline (speedup 1.0000x reference)
import jax
import jax.numpy as jnp
from jax.experimental import pallas as pl


def kernel(x, table, W, b):
    raise NotImplementedError("write your pallas kernel here")


# speedup vs baseline: 7.4022x; 7.4022x over previous
import functools

import jax
import jax.numpy as jnp
from jax import lax
from jax.experimental import pallas as pl
from jax.experimental.pallas import tpu as pltpu
from jax.experimental.pallas import tpu_sc as plsc

VOCAB = 10000
EMBED_DIM = 256
BATCH = 4096
HIST = 50

NUM_CORES = 2
NUM_SUBCORES = 16
NUM_WORKERS = NUM_CORES * NUM_SUBCORES
BPW = BATCH // NUM_WORKERS


def _pool_body(idx_hbm, table_hbm, out_hbm, idx_v, buf0, buf1, acc, sem0, sem1):
    c = lax.axis_index("c")
    s = lax.axis_index("s")
    wid = c * NUM_SUBCORES + s
    base = wid * BPW

    pltpu.sync_copy(idx_hbm.at[wid], idx_v)

    pltpu.async_copy(table_hbm.at[idx_v.at[0]], buf0, sem0).wait()
    pltpu.async_copy(table_hbm.at[idx_v.at[1]], buf1, sem1)

    def store_chunk(buf):
        def rbody(r, carry):
            for d in range(EMBED_DIM // 16):
                acc[r, pl.ds(d * 16, 16)] = buf[r, pl.ds(d * 16, 16)]
            return carry
        lax.fori_loop(0, BPW, rbody, 0)

    def add_chunk(buf):
        def rbody(r, carry):
            for d in range(EMBED_DIM // 16):
                plsc.addupdate(acc.at[r, pl.ds(d * 16, 16)],
                               buf[r, pl.ds(d * 16, 16)])
            return carry
        lax.fori_loop(0, BPW, rbody, 0)

    store_chunk(buf0)

    def body(k, carry):
        j = 2 * k + 1
        pltpu.make_async_copy(table_hbm.at[idx_v.at[j]], buf1, sem1).wait()
        pltpu.async_copy(table_hbm.at[idx_v.at[j + 1]], buf0, sem0)
        add_chunk(buf1)
        pltpu.make_async_copy(table_hbm.at[idx_v.at[j + 1]], buf0, sem0).wait()
        pltpu.async_copy(table_hbm.at[idx_v.at[j + 2]], buf1, sem1)
        add_chunk(buf0)
        return carry

    lax.fori_loop(0, (HIST - 2) // 2, body, 0)
    pltpu.make_async_copy(table_hbm.at[idx_v.at[HIST - 1]], buf1, sem1).wait()
    add_chunk(buf1)

    pltpu.sync_copy(acc, out_hbm.at[pl.ds(base, BPW)])


@functools.cache
def _pool():
    return pl.kernel(
        _pool_body,
        out_type=jax.ShapeDtypeStruct((BATCH, EMBED_DIM), jnp.float32),
        mesh=plsc.VectorSubcoreMesh(
            core_axis_name="c", subcore_axis_name="s",
            num_cores=NUM_CORES, num_subcores=NUM_SUBCORES,
        ),
        scratch_types=[
            pltpu.VMEM((HIST, BPW), jnp.int32),
            pltpu.VMEM((BPW, EMBED_DIM), jnp.float32),
            pltpu.VMEM((BPW, EMBED_DIM), jnp.float32),
            pltpu.VMEM((BPW, EMBED_DIM), jnp.float32),
            pltpu.SemaphoreType.DMA,
            pltpu.SemaphoreType.DMA,
        ],
    )


def _head_body(p_ref, w_ref, b_ref, o_ref):
    p = p_ref[...]
    h = lax.dot_general(
        p, w_ref[...], (((1,), (1,)), ((), ())),
        preferred_element_type=jnp.float32,
    )
    h = h * (1.0 / HIST) + b_ref[...]
    norm = jnp.sqrt(jnp.sum(h * h, axis=1, keepdims=True))
    o_ref[...] = h / jnp.maximum(norm, 1e-12)


def _head(pooled_sum, W, b2d):
    blk = 512
    grid = BATCH // blk
    return pl.pallas_call(
        _head_body,
        grid=(grid,),
        in_specs=[
            pl.BlockSpec((blk, EMBED_DIM), lambda i: (i, 0)),
            pl.BlockSpec((EMBED_DIM, EMBED_DIM), lambda i: (0, 0)),
            pl.BlockSpec((1, EMBED_DIM), lambda i: (0, 0)),
        ],
        out_specs=pl.BlockSpec((blk, EMBED_DIM), lambda i: (i, 0)),
        out_shape=jax.ShapeDtypeStruct((BATCH, EMBED_DIM), jnp.float32),
    )(pooled_sum, W, b2d)


@jax.jit
def kernel(x, table, W, b):
    idx = x.astype(jnp.int32).reshape(NUM_WORKERS, BPW, HIST)
    idx = jnp.transpose(idx, (0, 2, 1))
    pooled_sum = _pool()(idx, table)
    return _head(pooled_sum, W, b.reshape(1, EMBED_DIM))
